# baseline (speedup 1.0000x reference)
import functools

import jax
import jax.numpy as jnp
from jax import lax
from jax.experimental import pallas as pl
from jax.experimental.pallas import tpu as pltpu
from jax.experimental.pallas import tpu_sc as plsc

H = 384
W = 384
C = 96
B = 4
NC = 2
NS = 16
NW = NC * NS
ROWS_PER_W = (B * H) // NW
PIX_PER_W = ROWS_PER_W * W
CHUNK = 64
CP = 128
CPR = W // CHUNK
NCHUNKS = PIX_PER_W // CHUNK
NPAIR = NCHUNKS // 2
CG = C // 16

_GATHER_DNUMS = lax.GatherDimensionNumbers(
    offset_dims=(), collapsed_slice_dims=(0,), start_index_map=(0,))


def _bcast_lane(vec, lidx):
    return lax.gather(vec, lidx[:, None], _GATHER_DNUMS, (1,),
                      mode=lax.GatherScatterMode.PROMISE_IN_BOUNDS)


def _set_scratch():
    return [
        pltpu.VMEM((CHUNK,), jnp.float32),
        pltpu.VMEM((CHUNK,), jnp.float32),
        pltpu.VMEM((CHUNK,), jnp.int32),
        pltpu.VMEM((CHUNK,), jnp.int32),
        pltpu.VMEM((CHUNK,), jnp.int32),
        pltpu.VMEM((CHUNK,), jnp.int32),
        pltpu.VMEM((CHUNK,), jnp.float32),
        pltpu.VMEM((CHUNK,), jnp.float32),
        pltpu.VMEM((CHUNK,), jnp.float32),
        pltpu.VMEM((CHUNK,), jnp.float32),
        pltpu.VMEM((CHUNK, CP), jnp.float32),
        pltpu.VMEM((CHUNK, CP), jnp.float32),
        pltpu.VMEM((CHUNK, CP), jnp.float32),
        pltpu.VMEM((CHUNK, CP), jnp.float32),
        pltpu.VMEM((CHUNK, C), jnp.float32),
        pltpu.SemaphoreType.DMA,
        pltpu.SemaphoreType.DMA,
    ]


@functools.partial(
    pl.kernel,
    out_type=jax.ShapeDtypeStruct((B * H, W, C), jnp.float32),
    mesh=plsc.VectorSubcoreMesh(core_axis_name="c", subcore_axis_name="s"),
    compiler_params=pltpu.CompilerParams(
        needs_layout_passes=False, use_tc_tiling_on_sc=True),
    scratch_types=[pltpu.VMEM((16,), jnp.float32)] + _set_scratch() + _set_scratch(),
)
def _sc_bilinear(img_hbm, px_hbm, py_hbm, out_hbm, base_v, *scr):
    sets = [scr[:17], scr[17:]]
    wid = lax.axis_index("s") * NC + lax.axis_index("c")

    def stage_a(cc, s):
        (px_v, py_v, ia_v, ib_v, ic_v, id_v, wa_v, wb_v, wc_v, wd_v,
         ra_v, rb_v, rc_v, rd_v, _outc, gsem, _osem) = sets[s]
        c = wid + cc * NW
        gpix = c * CHUNK
        base_i = lax.div(gpix, H * W) * (H * W)
        pltpu.sync_copy(px_hbm.at[pl.ds(gpix, CHUNK)], px_v)
        pltpu.sync_copy(py_hbm.at[pl.ds(gpix, CHUNK)], py_v)
        act_v = jnp.zeros((16,), dtype=jnp.int32)
        for g in range(CHUNK // 16):
            sl = pl.ds(g * 16, 16)
            px = px_v[sl]
            py = py_v[sl]
            x0 = px.astype(jnp.int32)
            y0 = py.astype(jnp.int32)
            x1 = x0 + 1
            y1 = y0 + 1
            x0 = jnp.minimum(jnp.maximum(x0, 0), W - 1)
            x1 = jnp.minimum(jnp.maximum(x1, 0), W - 1)
            y0 = jnp.minimum(jnp.maximum(y0, 0), H - 1)
            y1 = jnp.minimum(jnp.maximum(y1, 0), H - 1)
            x0f = x0.astype(jnp.float32)
            x1f = x1.astype(jnp.float32)
            y0f = y0.astype(jnp.float32)
            y1f = y1.astype(jnp.float32)
            dx0 = px - x0f
            dx1 = x1f - px
            dy0 = py - y0f
            dy1 = y1f - py
            by0 = base_i + y0 * W
            by1 = base_i + y1 * W
            ia_v[sl] = by0 + x0
            ib_v[sl] = by1 + x0
            ic_v[sl] = by0 + x1
            id_v[sl] = by1 + x1
            wa_v[sl] = dx1 * dy1
            wb_v[sl] = dx1 * dy0
            wc_v[sl] = dx0 * dy1
            wd_v[sl] = dx0 * dy0
            inr = ((px > -1.0) & (px < jnp.float32(W - 1))
                   & (py > -1.0) & (py < jnp.float32(H - 1)))
            act_v = jnp.maximum(act_v, inr.astype(jnp.int32))
        pred = jnp.max(act_v) > 0

        @pl.when(pred)
        def _():
            pltpu.async_copy(img_hbm.at[ia_v], ra_v, gsem)
            pltpu.async_copy(img_hbm.at[ib_v], rb_v, gsem)
            pltpu.async_copy(img_hbm.at[ic_v], rc_v, gsem)
            pltpu.async_copy(img_hbm.at[id_v], rd_v, gsem)

        return pred

    def stage_b(cc, s, pred, not_first):
        (_px, _py, ia_v, ib_v, ic_v, id_v, wa_v, wb_v, wc_v, wd_v,
         ra_v, rb_v, rc_v, rd_v, outc_v, gsem, osem) = sets[s]

        @pl.when(pred)
        def _():
            pltpu.make_async_copy(img_hbm.at[ia_v], ra_v, gsem).wait()
            pltpu.make_async_copy(img_hbm.at[ib_v], rb_v, gsem).wait()
            pltpu.make_async_copy(img_hbm.at[ic_v], rc_v, gsem).wait()
            pltpu.make_async_copy(img_hbm.at[id_v], rd_v, gsem).wait()

        @pl.when(not_first)
        def _():
            pltpu.make_async_copy(
                outc_v, out_hbm.at[0, pl.ds(0, CHUNK), :], osem).wait()

        def grp_body(g, _):
            gbase = g * 16
            wa16 = wa_v[pl.ds(gbase, 16)]
            wb16 = wb_v[pl.ds(gbase, 16)]
            wc16 = wc_v[pl.ds(gbase, 16)]
            wd16 = wd_v[pl.ds(gbase, 16)]
            for l in range(16):
                lidx = jnp.full((16,), l, dtype=jnp.int32)
                wab = _bcast_lane(wa16, lidx)
                wbb = _bcast_lane(wb16, lidx)
                wcb = _bcast_lane(wc16, lidx)
                wdb = _bcast_lane(wd16, lidx)
                p = gbase + l
                for cg in range(CG):
                    csl = pl.ds(cg * 16, 16)
                    acc = wab * ra_v[p, csl] + wbb * rb_v[p, csl]
                    acc = acc + wcb * rc_v[p, csl] + wdb * rd_v[p, csl]
                    outc_v[p, csl] = acc
            return 0

        @pl.when(pred)
        def _():
            lax.fori_loop(0, CHUNK // 16, grp_body, 0)

        @pl.when(jnp.logical_not(pred))
        def _():
            zeros = jnp.zeros((16,), dtype=jnp.float32)

            def zero_body(p, _):
                for cg in range(CG):
                    outc_v[p, pl.ds(cg * 16, 16)] = zeros
                return 0

            lax.fori_loop(0, CHUNK, zero_body, 0)

        c = wid + cc * NW
        row = lax.div(c, CPR)
        j0 = lax.rem(c, CPR) * CHUNK
        pltpu.async_copy(outc_v, out_hbm.at[row, pl.ds(j0, CHUNK), :], osem)

    p0_init = stage_a(0, 0)

    def pair_body(cc2, p0):
        cc = 2 * cc2
        not_first = cc2 > 0
        p1 = stage_a(cc + 1, 1)
        stage_b(cc, 0, p0, not_first)

        p0n = lax.cond(cc2 < NPAIR - 1,
                       lambda: stage_a(cc + 2, 0),
                       lambda: jnp.bool_(False))
        stage_b(cc + 1, 1, p1, not_first)
        return p0n

    lax.fori_loop(0, NPAIR, pair_body, p0_init)

    pltpu.make_async_copy(
        sets[0][14], out_hbm.at[0, pl.ds(0, CHUNK), :], sets[0][16]).wait()
    pltpu.make_async_copy(
        sets[1][14], out_hbm.at[0, pl.ds(0, CHUNK), :], sets[1][16]).wait()


def kernel(X, transformation):
    batch, h, w, c = X.shape
    x_lin = jnp.linspace(-1.0, 1.0, w)
    y_lin = jnp.linspace(-1.0, 1.0, h)
    x_co, y_co = jnp.meshgrid(x_lin, y_lin)
    grid = jnp.concatenate(
        [x_co.ravel(), y_co.ravel(), jnp.ones_like(x_co.ravel())], axis=0)
    grids = jnp.tile(grid, (batch,)).reshape(batch, 3, h * w)
    sampled = jnp.einsum(
        'bij,bjk->bik', transformation.reshape(batch, 2, 3), grids)
    xs = sampled[:, 0:1, :].ravel().astype(jnp.float32)
    ys = sampled[:, 1:2, :].ravel().astype(jnp.float32)
    px = 0.5 * (xs + 1.0) * jnp.float32(w)
    py = 0.5 * (ys + 1.0) * jnp.float32(h)

    flat_img = jnp.pad(
        X.reshape(batch * h * w, c).astype(jnp.float32), ((0, 0), (0, CP - c)))
    out = _sc_bilinear(flat_img, px, py)
    return out.reshape(batch, h, w, c)


# speedup vs baseline: 1.1387x; 1.1387x over previous
import functools

import jax
import jax.numpy as jnp
from jax import lax
from jax.experimental import pallas as pl
from jax.experimental.pallas import tpu as pltpu
from jax.experimental.pallas import tpu_sc as plsc

H = 384
W = 384
C = 96
B = 4
NC = 2
NS = 16
NW = NC * NS
ROWS_PER_W = (B * H) // NW
PIX_PER_W = ROWS_PER_W * W
CHUNK = 64
CP = 128
CPR = W // CHUNK
NCHUNKS = PIX_PER_W // CHUNK
NPAIR = NCHUNKS // 2
CG = C // 16

_GATHER_DNUMS = lax.GatherDimensionNumbers(
    offset_dims=(), collapsed_slice_dims=(0,), start_index_map=(0,))


def _bcast_lane(vec, lidx):
    return lax.gather(vec, lidx[:, None], _GATHER_DNUMS, (1,),
                      mode=lax.GatherScatterMode.PROMISE_IN_BOUNDS)


def _set_scratch():
    return [
        pltpu.VMEM((CHUNK,), jnp.float32),
        pltpu.VMEM((CHUNK,), jnp.float32),
        pltpu.VMEM((CHUNK,), jnp.int32),
        pltpu.VMEM((CHUNK,), jnp.int32),
        pltpu.VMEM((CHUNK,), jnp.int32),
        pltpu.VMEM((CHUNK,), jnp.int32),
        pltpu.VMEM((CHUNK,), jnp.float32),
        pltpu.VMEM((CHUNK,), jnp.float32),
        pltpu.VMEM((CHUNK,), jnp.float32),
        pltpu.VMEM((CHUNK,), jnp.float32),
        pltpu.VMEM((CHUNK, CP), jnp.float32),
        pltpu.VMEM((CHUNK, CP), jnp.float32),
        pltpu.VMEM((CHUNK, CP), jnp.float32),
        pltpu.VMEM((CHUNK, CP), jnp.float32),
        pltpu.VMEM((CHUNK, C), jnp.float32),
        pltpu.SemaphoreType.DMA,
        pltpu.SemaphoreType.DMA,
    ]


@functools.partial(
    pl.kernel,
    out_type=jax.ShapeDtypeStruct((B * H, W, C), jnp.float32),
    mesh=plsc.VectorSubcoreMesh(core_axis_name="c", subcore_axis_name="s"),
    compiler_params=pltpu.CompilerParams(
        needs_layout_passes=False, use_tc_tiling_on_sc=True),
    scratch_types=[pltpu.VMEM((16,), jnp.float32)] + _set_scratch() + _set_scratch(),
)
def _sc_bilinear(img_hbm, px_hbm, py_hbm, out_hbm, base_v, *scr):
    sets = [scr[:17], scr[17:]]
    wid = lax.axis_index("s") * NC + lax.axis_index("c")

    def stage_a(cc, s):
        (px_v, py_v, ia_v, ib_v, ic_v, id_v, wa_v, wb_v, wc_v, wd_v,
         ra_v, rb_v, rc_v, rd_v, _outc, gsem, _osem) = sets[s]
        c = wid * NCHUNKS + cc
        gpix = c * CHUNK
        base_i = lax.div(gpix, H * W) * (H * W)
        pltpu.sync_copy(px_hbm.at[pl.ds(gpix, CHUNK)], px_v)
        pltpu.sync_copy(py_hbm.at[pl.ds(gpix, CHUNK)], py_v)
        act_v = jnp.zeros((16,), dtype=jnp.int32)
        for g in range(CHUNK // 16):
            sl = pl.ds(g * 16, 16)
            px = px_v[sl]
            py = py_v[sl]
            x0 = px.astype(jnp.int32)
            y0 = py.astype(jnp.int32)
            x1 = x0 + 1
            y1 = y0 + 1
            x0 = jnp.minimum(jnp.maximum(x0, 0), W - 1)
            x1 = jnp.minimum(jnp.maximum(x1, 0), W - 1)
            y0 = jnp.minimum(jnp.maximum(y0, 0), H - 1)
            y1 = jnp.minimum(jnp.maximum(y1, 0), H - 1)
            x0f = x0.astype(jnp.float32)
            x1f = x1.astype(jnp.float32)
            y0f = y0.astype(jnp.float32)
            y1f = y1.astype(jnp.float32)
            dx0 = px - x0f
            dx1 = x1f - px
            dy0 = py - y0f
            dy1 = y1f - py
            by0 = base_i + y0 * W
            by1 = base_i + y1 * W
            ia_v[sl] = by0 + x0
            ib_v[sl] = by1 + x0
            ic_v[sl] = by0 + x1
            id_v[sl] = by1 + x1
            wa_v[sl] = dx1 * dy1
            wb_v[sl] = dx1 * dy0
            wc_v[sl] = dx0 * dy1
            wd_v[sl] = dx0 * dy0
            inr = ((px > -1.0) & (px < jnp.float32(W - 1))
                   & (py > -1.0) & (py < jnp.float32(H - 1)))
            act_v = jnp.maximum(act_v, inr.astype(jnp.int32))
        pred = jnp.max(act_v) > 0

        @pl.when(pred)
        def _():
            pltpu.async_copy(img_hbm.at[ia_v], ra_v, gsem)
            pltpu.async_copy(img_hbm.at[ib_v], rb_v, gsem)
            pltpu.async_copy(img_hbm.at[ic_v], rc_v, gsem)
            pltpu.async_copy(img_hbm.at[id_v], rd_v, gsem)

        return pred

    def stage_b(cc, s, pred, not_first):
        (_px, _py, ia_v, ib_v, ic_v, id_v, wa_v, wb_v, wc_v, wd_v,
         ra_v, rb_v, rc_v, rd_v, outc_v, gsem, osem) = sets[s]

        @pl.when(pred)
        def _():
            pltpu.make_async_copy(img_hbm.at[ia_v], ra_v, gsem).wait()
            pltpu.make_async_copy(img_hbm.at[ib_v], rb_v, gsem).wait()
            pltpu.make_async_copy(img_hbm.at[ic_v], rc_v, gsem).wait()
            pltpu.make_async_copy(img_hbm.at[id_v], rd_v, gsem).wait()

        @pl.when(not_first)
        def _():
            pltpu.make_async_copy(
                outc_v, out_hbm.at[0, pl.ds(0, CHUNK), :], osem).wait()

        def grp_body(g, _):
            gbase = g * 16
            wa16 = wa_v[pl.ds(gbase, 16)]
            wb16 = wb_v[pl.ds(gbase, 16)]
            wc16 = wc_v[pl.ds(gbase, 16)]
            wd16 = wd_v[pl.ds(gbase, 16)]
            for l in range(16):
                lidx = jnp.full((16,), l, dtype=jnp.int32)
                wab = _bcast_lane(wa16, lidx)
                wbb = _bcast_lane(wb16, lidx)
                wcb = _bcast_lane(wc16, lidx)
                wdb = _bcast_lane(wd16, lidx)
                p = gbase + l
                for cg in range(CG):
                    csl = pl.ds(cg * 16, 16)
                    acc = wab * ra_v[p, csl] + wbb * rb_v[p, csl]
                    acc = acc + wcb * rc_v[p, csl] + wdb * rd_v[p, csl]
                    outc_v[p, csl] = acc
            return 0

        @pl.when(pred)
        def _():
            lax.fori_loop(0, CHUNK // 16, grp_body, 0)

        @pl.when(jnp.logical_not(pred))
        def _():
            zeros = jnp.zeros((16,), dtype=jnp.float32)

            def zero_body(p, _):
                for cg in range(CG):
                    outc_v[p, pl.ds(cg * 16, 16)] = zeros
                return 0

            lax.fori_loop(0, CHUNK, zero_body, 0)

        c = wid * NCHUNKS + cc
        row = lax.div(c, CPR)
        j0 = lax.rem(c, CPR) * CHUNK
        pltpu.async_copy(outc_v, out_hbm.at[row, pl.ds(j0, CHUNK), :], osem)

    p0_init = stage_a(0, 0)

    def pair_body(cc2, p0):
        cc = 2 * cc2
        not_first = cc2 > 0
        p1 = stage_a(cc + 1, 1)
        stage_b(cc, 0, p0, not_first)

        p0n = lax.cond(cc2 < NPAIR - 1,
                       lambda: stage_a(cc + 2, 0),
                       lambda: jnp.bool_(False))
        stage_b(cc + 1, 1, p1, not_first)
        return p0n

    lax.fori_loop(0, NPAIR, pair_body, p0_init)

    pltpu.make_async_copy(
        sets[0][14], out_hbm.at[0, pl.ds(0, CHUNK), :], sets[0][16]).wait()
    pltpu.make_async_copy(
        sets[1][14], out_hbm.at[0, pl.ds(0, CHUNK), :], sets[1][16]).wait()


def kernel(X, transformation):
    batch, h, w, c = X.shape
    x_lin = jnp.linspace(-1.0, 1.0, w)
    y_lin = jnp.linspace(-1.0, 1.0, h)
    x_co, y_co = jnp.meshgrid(x_lin, y_lin)
    grid = jnp.concatenate(
        [x_co.ravel(), y_co.ravel(), jnp.ones_like(x_co.ravel())], axis=0)
    grids = jnp.tile(grid, (batch,)).reshape(batch, 3, h * w)
    sampled = jnp.einsum(
        'bij,bjk->bik', transformation.reshape(batch, 2, 3), grids)
    xs = sampled[:, 0:1, :].ravel().astype(jnp.float32)
    ys = sampled[:, 1:2, :].ravel().astype(jnp.float32)
    px = 0.5 * (xs + 1.0) * jnp.float32(w)
    py = 0.5 * (ys + 1.0) * jnp.float32(h)

    flat_img = jnp.pad(
        X.reshape(batch * h * w, c).astype(jnp.float32), ((0, 0), (0, CP - c)))
    out = _sc_bilinear(flat_img, px, py)
    return out.reshape(batch, h, w, c)


# speedup vs baseline: 1.1403x; 1.0013x over previous
import functools

import jax
import jax.numpy as jnp
from jax import lax
from jax.experimental import pallas as pl
from jax.experimental.pallas import tpu as pltpu
from jax.experimental.pallas import tpu_sc as plsc

H = 384
W = 384
C = 96
B = 4
NC = 2
NS = 16
NW = NC * NS
CHUNK = 64
CP = 128
CPR = W // CHUNK
NCHUNKS = (B * H * W) // (NW * CHUNK)
NPAIR = NCHUNKS // 2
CG = C // 16

_GATHER_DNUMS = lax.GatherDimensionNumbers(
    offset_dims=(), collapsed_slice_dims=(0,), start_index_map=(0,))


def _bcast_lane(vec, lidx):
    return lax.gather(vec, lidx[:, None], _GATHER_DNUMS, (1,),
                      mode=lax.GatherScatterMode.PROMISE_IN_BOUNDS)


def _set_scratch():
    return [
        pltpu.VMEM((CHUNK,), jnp.float32),
        pltpu.VMEM((CHUNK,), jnp.float32),
        pltpu.VMEM((CHUNK,), jnp.int32),
        pltpu.VMEM((CHUNK,), jnp.int32),
        pltpu.VMEM((CHUNK,), jnp.int32),
        pltpu.VMEM((CHUNK,), jnp.int32),
        pltpu.VMEM((CHUNK,), jnp.float32),
        pltpu.VMEM((CHUNK,), jnp.float32),
        pltpu.VMEM((CHUNK,), jnp.float32),
        pltpu.VMEM((CHUNK,), jnp.float32),
        pltpu.VMEM((CHUNK, CP), jnp.float32),
        pltpu.VMEM((CHUNK, CP), jnp.float32),
        pltpu.VMEM((CHUNK, CP), jnp.float32),
        pltpu.VMEM((CHUNK, CP), jnp.float32),
        pltpu.VMEM((CHUNK, C), jnp.float32),
        pltpu.SemaphoreType.DMA,
        pltpu.SemaphoreType.DMA,
    ]


@functools.partial(
    pl.kernel,
    out_type=jax.ShapeDtypeStruct((B * H, W, C), jnp.float32),
    mesh=plsc.VectorSubcoreMesh(core_axis_name="c", subcore_axis_name="s"),
    compiler_params=pltpu.CompilerParams(
        needs_layout_passes=False, use_tc_tiling_on_sc=True),
    scratch_types=_set_scratch() + _set_scratch(),
)
def _sc_bilinear(img_hbm, px_hbm, py_hbm, out_hbm, *scr):
    sets = [scr[:17], scr[17:]]
    wid = lax.axis_index("s") * NC + lax.axis_index("c")

    def stage_a(cc, s):
        (px_v, py_v, ia_v, ib_v, ic_v, id_v, wa_v, wb_v, wc_v, wd_v,
         ra_v, rb_v, rc_v, rd_v, _outc, gsem, _osem) = sets[s]
        c = wid * NCHUNKS + cc
        gpix = c * CHUNK
        base_i = lax.div(gpix, H * W) * (H * W)
        pltpu.sync_copy(px_hbm.at[pl.ds(gpix, CHUNK)], px_v)
        pltpu.sync_copy(py_hbm.at[pl.ds(gpix, CHUNK)], py_v)
        act_v = jnp.zeros((16,), dtype=jnp.int32)
        for g in range(CHUNK // 16):
            sl = pl.ds(g * 16, 16)
            px = px_v[sl]
            py = py_v[sl]
            x0 = px.astype(jnp.int32)
            y0 = py.astype(jnp.int32)
            x1 = x0 + 1
            y1 = y0 + 1
            x0 = jnp.minimum(jnp.maximum(x0, 0), W - 1)
            x1 = jnp.minimum(jnp.maximum(x1, 0), W - 1)
            y0 = jnp.minimum(jnp.maximum(y0, 0), H - 1)
            y1 = jnp.minimum(jnp.maximum(y1, 0), H - 1)
            x0f = x0.astype(jnp.float32)
            x1f = x1.astype(jnp.float32)
            y0f = y0.astype(jnp.float32)
            y1f = y1.astype(jnp.float32)
            dx0 = px - x0f
            dx1 = x1f - px
            dy0 = py - y0f
            dy1 = y1f - py
            by0 = base_i + y0 * W
            by1 = base_i + y1 * W
            ia_v[sl] = by0 + x0
            ib_v[sl] = by1 + x0
            ic_v[sl] = by0 + x1
            id_v[sl] = by1 + x1
            wa_v[sl] = dx1 * dy1
            wb_v[sl] = dx1 * dy0
            wc_v[sl] = dx0 * dy1
            wd_v[sl] = dx0 * dy0
            inr = ((px > -1.0) & (px < jnp.float32(W - 1))
                   & (py > -1.0) & (py < jnp.float32(H - 1)))
            act_v = jnp.maximum(act_v, inr.astype(jnp.int32))
        pred = jnp.max(act_v) > 0

        @pl.when(pred)
        def _():
            pltpu.async_copy(img_hbm.at[ia_v], ra_v, gsem)
            pltpu.async_copy(img_hbm.at[ib_v], rb_v, gsem)
            pltpu.async_copy(img_hbm.at[ic_v], rc_v, gsem)
            pltpu.async_copy(img_hbm.at[id_v], rd_v, gsem)

        return pred

    def stage_b(cc, s, pred, not_first):
        (_px, _py, ia_v, ib_v, ic_v, id_v, wa_v, wb_v, wc_v, wd_v,
         ra_v, rb_v, rc_v, rd_v, outc_v, gsem, osem) = sets[s]

        @pl.when(pred)
        def _():
            pltpu.make_async_copy(img_hbm.at[ia_v], ra_v, gsem).wait()
            pltpu.make_async_copy(img_hbm.at[ib_v], rb_v, gsem).wait()
            pltpu.make_async_copy(img_hbm.at[ic_v], rc_v, gsem).wait()
            pltpu.make_async_copy(img_hbm.at[id_v], rd_v, gsem).wait()

        @pl.when(not_first)
        def _():
            pltpu.make_async_copy(
                outc_v, out_hbm.at[0, pl.ds(0, CHUNK), :], osem).wait()

        def grp_body(g, _):
            gbase = g * 16
            wa16 = wa_v[pl.ds(gbase, 16)]
            wb16 = wb_v[pl.ds(gbase, 16)]
            wc16 = wc_v[pl.ds(gbase, 16)]
            wd16 = wd_v[pl.ds(gbase, 16)]
            for l in range(16):
                lidx = jnp.full((16,), l, dtype=jnp.int32)
                wab = _bcast_lane(wa16, lidx)
                wbb = _bcast_lane(wb16, lidx)
                wcb = _bcast_lane(wc16, lidx)
                wdb = _bcast_lane(wd16, lidx)
                p = gbase + l
                for cg in range(CG):
                    csl = pl.ds(cg * 16, 16)
                    acc = wab * ra_v[p, csl] + wbb * rb_v[p, csl]
                    acc = acc + wcb * rc_v[p, csl] + wdb * rd_v[p, csl]
                    outc_v[p, csl] = acc
            return 0

        @pl.when(pred)
        def _():
            lax.fori_loop(0, CHUNK // 16, grp_body, 0)

        @pl.when(jnp.logical_not(pred))
        def _():
            zeros = jnp.zeros((16,), dtype=jnp.float32)

            def zero_body(p, _):
                for cg in range(CG):
                    outc_v[p, pl.ds(cg * 16, 16)] = zeros
                return 0

            lax.fori_loop(0, CHUNK, zero_body, 0)

        c = wid * NCHUNKS + cc
        row = lax.div(c, CPR)
        j0 = lax.rem(c, CPR) * CHUNK
        pltpu.async_copy(outc_v, out_hbm.at[row, pl.ds(j0, CHUNK), :], osem)

    p0_init = stage_a(0, 0)

    def pair_body(cc2, p0):
        cc = 2 * cc2
        not_first = cc2 > 0
        p1 = stage_a(cc + 1, 1)
        stage_b(cc, 0, p0, not_first)

        p0n = lax.cond(cc2 < NPAIR - 1,
                       lambda: stage_a(cc + 2, 0),
                       lambda: jnp.bool_(False))
        stage_b(cc + 1, 1, p1, not_first)
        return p0n

    lax.fori_loop(0, NPAIR, pair_body, p0_init)

    pltpu.make_async_copy(
        sets[0][14], out_hbm.at[0, pl.ds(0, CHUNK), :], sets[0][16]).wait()
    pltpu.make_async_copy(
        sets[1][14], out_hbm.at[0, pl.ds(0, CHUNK), :], sets[1][16]).wait()


def kernel(X, transformation):
    batch, h, w, c = X.shape
    x_lin = jnp.linspace(-1.0, 1.0, w)
    y_lin = jnp.linspace(-1.0, 1.0, h)
    x_co, y_co = jnp.meshgrid(x_lin, y_lin)
    grid = jnp.concatenate(
        [x_co.ravel(), y_co.ravel(), jnp.ones_like(x_co.ravel())], axis=0)
    grids = jnp.tile(grid, (batch,)).reshape(batch, 3, h * w)
    sampled = jnp.einsum(
        'bij,bjk->bik', transformation.reshape(batch, 2, 3), grids)
    xs = sampled[:, 0:1, :].ravel().astype(jnp.float32)
    ys = sampled[:, 1:2, :].ravel().astype(jnp.float32)
    px = 0.5 * (xs + 1.0) * jnp.float32(w)
    py = 0.5 * (ys + 1.0) * jnp.float32(h)

    flat_img = jnp.pad(
        X.reshape(batch * h * w, c).astype(jnp.float32), ((0, 0), (0, CP - c)))
    out = _sc_bilinear(flat_img, px, py)
    return out.reshape(batch, h, w, c)


# speedup vs baseline: 1.1752x; 1.0306x over previous
import functools

import jax
import jax.numpy as jnp
from jax import lax
from jax.experimental import pallas as pl
from jax.experimental.pallas import tpu as pltpu
from jax.experimental.pallas import tpu_sc as plsc

H = 384
W = 384
C = 96
B = 4
NC = 2
NS = 16
NW = NC * NS
CHUNK = 64
CP = 128
CPR = W // CHUNK
NCHUNKS = (B * H * W) // (NW * CHUNK)
NPAIR = NCHUNKS // 2
CG = C // 16

_GATHER_DNUMS = lax.GatherDimensionNumbers(
    offset_dims=(), collapsed_slice_dims=(0,), start_index_map=(0,))


def _bcast_lane(vec, lidx):
    return lax.gather(vec, lidx[:, None], _GATHER_DNUMS, (1,),
                      mode=lax.GatherScatterMode.PROMISE_IN_BOUNDS)


def _set_scratch():
    return [
        pltpu.VMEM((2 * CHUNK,), jnp.float32),
        pltpu.VMEM((CHUNK,), jnp.int32),
        pltpu.VMEM((CHUNK,), jnp.int32),
        pltpu.VMEM((CHUNK,), jnp.int32),
        pltpu.VMEM((CHUNK,), jnp.int32),
        pltpu.VMEM((CHUNK,), jnp.float32),
        pltpu.VMEM((CHUNK,), jnp.float32),
        pltpu.VMEM((CHUNK,), jnp.float32),
        pltpu.VMEM((CHUNK,), jnp.float32),
        pltpu.VMEM((CHUNK, CP), jnp.float32),
        pltpu.VMEM((CHUNK, CP), jnp.float32),
        pltpu.VMEM((CHUNK, CP), jnp.float32),
        pltpu.VMEM((CHUNK, CP), jnp.float32),
        pltpu.VMEM((CHUNK, C), jnp.float32),
        pltpu.SemaphoreType.DMA,
        pltpu.SemaphoreType.DMA,
    ]


@functools.partial(
    pl.kernel,
    out_type=jax.ShapeDtypeStruct((B * H, W, C), jnp.float32),
    mesh=plsc.VectorSubcoreMesh(core_axis_name="c", subcore_axis_name="s"),
    compiler_params=pltpu.CompilerParams(
        needs_layout_passes=False, use_tc_tiling_on_sc=True),
    scratch_types=_set_scratch() + _set_scratch(),
)
def _sc_bilinear(img_hbm, pxy_hbm, out_hbm, *scr):
    sets = [scr[:16], scr[16:]]
    wid = lax.axis_index("s") * NC + lax.axis_index("c")

    def stage_a(cc, s):
        (pxy_v, ia_v, ib_v, ic_v, id_v, wa_v, wb_v, wc_v, wd_v,
         ra_v, rb_v, rc_v, rd_v, _outc, gsem, _osem) = sets[s]
        c = wid * NCHUNKS + cc
        gpix = c * CHUNK
        base_i = lax.div(gpix, H * W) * (H * W)
        pltpu.sync_copy(pxy_hbm.at[c], pxy_v)
        act_v = jnp.zeros((16,), dtype=jnp.int32)
        for g in range(CHUNK // 16):
            sl = pl.ds(g * 16, 16)
            px = pxy_v[pl.ds(g * 16, 16)]
            py = pxy_v[pl.ds(CHUNK + g * 16, 16)]
            x0 = px.astype(jnp.int32)
            y0 = py.astype(jnp.int32)
            x1 = x0 + 1
            y1 = y0 + 1
            x0 = jnp.minimum(jnp.maximum(x0, 0), W - 1)
            x1 = jnp.minimum(jnp.maximum(x1, 0), W - 1)
            y0 = jnp.minimum(jnp.maximum(y0, 0), H - 1)
            y1 = jnp.minimum(jnp.maximum(y1, 0), H - 1)
            x0f = x0.astype(jnp.float32)
            x1f = x1.astype(jnp.float32)
            y0f = y0.astype(jnp.float32)
            y1f = y1.astype(jnp.float32)
            dx0 = px - x0f
            dx1 = x1f - px
            dy0 = py - y0f
            dy1 = y1f - py
            by0 = base_i + y0 * W
            by1 = base_i + y1 * W
            ia_v[sl] = by0 + x0
            ib_v[sl] = by1 + x0
            ic_v[sl] = by0 + x1
            id_v[sl] = by1 + x1
            wa_v[sl] = dx1 * dy1
            wb_v[sl] = dx1 * dy0
            wc_v[sl] = dx0 * dy1
            wd_v[sl] = dx0 * dy0
            inr = ((px > -1.0) & (px < jnp.float32(W - 1))
                   & (py > -1.0) & (py < jnp.float32(H - 1)))
            act_v = jnp.maximum(act_v, inr.astype(jnp.int32))
        pred = jnp.max(act_v) > 0

        @pl.when(pred)
        def _():
            pltpu.async_copy(img_hbm.at[ia_v], ra_v, gsem)
            pltpu.async_copy(img_hbm.at[ib_v], rb_v, gsem)
            pltpu.async_copy(img_hbm.at[ic_v], rc_v, gsem)
            pltpu.async_copy(img_hbm.at[id_v], rd_v, gsem)

        return pred

    def stage_b(cc, s, pred, not_first):
        (_pxy, ia_v, ib_v, ic_v, id_v, wa_v, wb_v, wc_v, wd_v,
         ra_v, rb_v, rc_v, rd_v, outc_v, gsem, osem) = sets[s]

        @pl.when(pred)
        def _():
            pltpu.make_async_copy(img_hbm.at[ia_v], ra_v, gsem).wait()
            pltpu.make_async_copy(img_hbm.at[ib_v], rb_v, gsem).wait()
            pltpu.make_async_copy(img_hbm.at[ic_v], rc_v, gsem).wait()
            pltpu.make_async_copy(img_hbm.at[id_v], rd_v, gsem).wait()

        @pl.when(not_first)
        def _():
            pltpu.make_async_copy(
                outc_v, out_hbm.at[0, pl.ds(0, CHUNK), :], osem).wait()

        def grp_body(g, _):
            gbase = g * 16
            wa16 = wa_v[pl.ds(gbase, 16)]
            wb16 = wb_v[pl.ds(gbase, 16)]
            wc16 = wc_v[pl.ds(gbase, 16)]
            wd16 = wd_v[pl.ds(gbase, 16)]
            for l in range(16):
                lidx = jnp.full((16,), l, dtype=jnp.int32)
                wab = _bcast_lane(wa16, lidx)
                wbb = _bcast_lane(wb16, lidx)
                wcb = _bcast_lane(wc16, lidx)
                wdb = _bcast_lane(wd16, lidx)
                p = gbase + l
                for cg in range(CG):
                    csl = pl.ds(cg * 16, 16)
                    acc = wab * ra_v[p, csl] + wbb * rb_v[p, csl]
                    acc = acc + wcb * rc_v[p, csl] + wdb * rd_v[p, csl]
                    outc_v[p, csl] = acc
            return 0

        @pl.when(pred)
        def _():
            lax.fori_loop(0, CHUNK // 16, grp_body, 0)

        @pl.when(jnp.logical_not(pred))
        def _():
            zeros = jnp.zeros((16,), dtype=jnp.float32)

            def zero_body(p, _):
                for cg in range(CG):
                    outc_v[p, pl.ds(cg * 16, 16)] = zeros
                return 0

            lax.fori_loop(0, CHUNK, zero_body, 0)

        c = wid * NCHUNKS + cc
        row = lax.div(c, CPR)
        j0 = lax.rem(c, CPR) * CHUNK
        pltpu.async_copy(outc_v, out_hbm.at[row, pl.ds(j0, CHUNK), :], osem)

    p0_init = stage_a(0, 0)

    def pair_body(cc2, p0):
        cc = 2 * cc2
        not_first = cc2 > 0
        p1 = stage_a(cc + 1, 1)
        stage_b(cc, 0, p0, not_first)

        p0n = lax.cond(cc2 < NPAIR - 1,
                       lambda: stage_a(cc + 2, 0),
                       lambda: jnp.bool_(False))
        stage_b(cc + 1, 1, p1, not_first)
        return p0n

    lax.fori_loop(0, NPAIR, pair_body, p0_init)

    pltpu.make_async_copy(
        sets[0][13], out_hbm.at[0, pl.ds(0, CHUNK), :], sets[0][15]).wait()
    pltpu.make_async_copy(
        sets[1][13], out_hbm.at[0, pl.ds(0, CHUNK), :], sets[1][15]).wait()


def kernel(X, transformation):
    batch, h, w, c = X.shape
    x_lin = jnp.linspace(-1.0, 1.0, w)
    y_lin = jnp.linspace(-1.0, 1.0, h)
    x_co, y_co = jnp.meshgrid(x_lin, y_lin)
    grid = jnp.concatenate(
        [x_co.ravel(), y_co.ravel(), jnp.ones_like(x_co.ravel())], axis=0)
    grids = jnp.tile(grid, (batch,)).reshape(batch, 3, h * w)
    sampled = jnp.einsum(
        'bij,bjk->bik', transformation.reshape(batch, 2, 3), grids)
    xs = sampled[:, 0:1, :].ravel().astype(jnp.float32)
    ys = sampled[:, 1:2, :].ravel().astype(jnp.float32)
    px = 0.5 * (xs + 1.0) * jnp.float32(w)
    py = 0.5 * (ys + 1.0) * jnp.float32(h)

    pxy = jnp.concatenate(
        [px.reshape(-1, CHUNK), py.reshape(-1, CHUNK)], axis=1)
    flat_img = jnp.pad(
        X.reshape(batch * h * w, c).astype(jnp.float32), ((0, 0), (0, CP - c)))
    out = _sc_bilinear(flat_img, pxy)
    return out.reshape(batch, h, w, c)
